# baseline (speedup 1.0000x reference)
import functools

import jax
import jax.numpy as jnp
from jax import lax
from jax.experimental import pallas as pl
from jax.experimental.pallas import tpu as pltpu
from jax.experimental.pallas import tpu_sc as plsc

D = 64
NC = 2
NS = 16
NW = NC * NS
CHUNK = 128


def _sc_gather(table, idx3):
    nw, nchunk, chunk = idx3.shape
    n = nw * nchunk * chunk
    per_w = nchunk * chunk
    mesh = plsc.VectorSubcoreMesh(core_axis_name="c", subcore_axis_name="s")

    @functools.partial(
        pl.kernel,
        out_type=jax.ShapeDtypeStruct((n, D), jnp.float32),
        mesh=mesh,
        scratch_types=[
            pltpu.VMEM((nchunk, chunk), jnp.int32),
            pltpu.VMEM((chunk, D), jnp.float32),
            pltpu.SemaphoreType.DMA,
        ],
        compiler_params=pltpu.CompilerParams(use_tc_tiling_on_sc=False),
    )
    def k(table_hbm, idx_hbm, out_hbm, idx_v, buf, sem):
        wid = lax.axis_index("s") * NC + lax.axis_index("c")
        base = wid * per_w
        pltpu.sync_copy(idx_hbm.at[wid], idx_v)

        @pl.loop(0, nchunk)
        def _(i):
            pltpu.async_copy(table_hbm.at[idx_v.at[i]], buf, sem).wait()
            pltpu.sync_copy(buf, out_hbm.at[pl.ds(base + i * chunk, chunk)])

    return k(table, idx3)


def _tc_mlp(x, W1, b1, W2, b2):
    n, d = x.shape
    blk = 4096

    def body(x_ref, w1_ref, b1_ref, w2_ref, b2_ref, o_ref):
        h = jnp.maximum(
            jnp.dot(x_ref[...], w1_ref[...], preferred_element_type=jnp.float32)
            + b1_ref[...], 0.0)
        o_ref[...] = (
            jnp.dot(h, w2_ref[...], preferred_element_type=jnp.float32)
            + b2_ref[...])

    return pl.pallas_call(
        body,
        grid=(n // blk,),
        in_specs=[
            pl.BlockSpec((blk, d), lambda i: (i, 0)),
            pl.BlockSpec((d, d), lambda i: (0, 0)),
            pl.BlockSpec((1, d), lambda i: (0, 0)),
            pl.BlockSpec((d, d), lambda i: (0, 0)),
            pl.BlockSpec((1, d), lambda i: (0, 0)),
        ],
        out_specs=pl.BlockSpec((blk, d), lambda i: (i, 0)),
        out_shape=jax.ShapeDtypeStruct((n, d), jnp.float32),
    )(x, W1, b1.reshape(1, d), W2, b2.reshape(1, d))


def kernel(class_indices, embedding, W1, b1, W2, b2):
    if class_indices.ndim == 1:
        class_indices = class_indices[:, None]
    q, b = class_indices.shape
    n = q * b
    per_w = n // NW
    nchunk = per_w // CHUNK
    idx3 = class_indices.reshape(NW, nchunk, CHUNK).astype(jnp.int32)
    gathered = _sc_gather(embedding, idx3)
    out = _tc_mlp(gathered, W1, b1, W2, b2)
    return out.reshape(q, b, D)


# speedup vs baseline: 1.3177x; 1.3177x over previous
import functools

import jax
import jax.numpy as jnp
from jax import lax
from jax.experimental import pallas as pl
from jax.experimental.pallas import tpu as pltpu
from jax.experimental.pallas import tpu_sc as plsc

D = 64
NC = 2
NS = 16
NW = NC * NS
GROUP = 16
NGROUP = 8


def _sc_gather(emb, idx3):
    V, _ = emb.shape
    nw, R, C = idx3.shape
    per_w = R * C
    n = nw * per_w
    ngrp = per_w // GROUP
    mesh = plsc.VectorSubcoreMesh(core_axis_name="c", subcore_axis_name="s")

    @functools.partial(
        pl.kernel,
        out_type=jax.ShapeDtypeStruct((n, D), jnp.float32),
        mesh=mesh,
        scratch_types=[
            pltpu.VMEM((R, C), jnp.int32),
            pltpu.VMEM((2 * GROUP, 8, D), jnp.float32),
            pltpu.VMEM((2, GROUP, D), jnp.float32),
            pltpu.SemaphoreType.DMA,
            pltpu.SemaphoreType.DMA,
            pltpu.SemaphoreType.DMA,
            pltpu.SemaphoreType.DMA,
        ],
    )
    def k(emb_hbm, idx_hbm, out_hbm, idx_v, slabs, rows, s_e, s_o, o_e, o_o):
        wid = lax.axis_index("s") * NC + lax.axis_index("c")
        base = wid * per_w
        pltpu.sync_copy(idx_hbm.at[wid], idx_v)

        def load_grp(g):
            return idx_v[g // NGROUP, pl.ds((g % NGROUP) * GROUP, GROUP)]

        def fire(g, par, sem):
            v = load_grp(g)
            half = par * GROUP
            for nn in range(GROUP):
                start = pl.multiple_of(v[nn] & jnp.int32(-8), 8)
                pltpu.async_copy(
                    emb_hbm.at[pl.ds(start, 8)],
                    slabs.at[half + nn], sem)

        def drain_extract(g, par, sem):
            v = load_grp(g)
            half = par * GROUP
            for nn in range(GROUP):
                pltpu.make_async_copy(
                    emb_hbm.at[pl.ds(0, 8)],
                    slabs.at[half + nn], sem).wait()
            for nn in range(GROUP):
                r = v[nn] & 7
                for j in range(D // 16):
                    rows[par, nn, pl.ds(16 * j, 16)] = (
                        slabs[half + nn, r, pl.ds(16 * j, 16)])

        def write_out(g, par, sem):
            pltpu.async_copy(
                rows.at[par],
                out_hbm.at[pl.ds(base + g * GROUP, GROUP)], sem)

        def wait_out(par, sem):
            pltpu.make_async_copy(
                rows.at[par],
                out_hbm.at[pl.ds(base, GROUP)], sem).wait()

        fire(0, 0, s_e)

        @pl.loop(0, ngrp // 2)
        def _(p):
            g = 2 * p

            @pl.when(g >= 2)
            def _():
                wait_out(0, o_e)
            fire(g + 1, 1, s_o)
            drain_extract(g, 0, s_e)
            write_out(g, 0, o_e)

            @pl.when(g >= 1)
            def _():
                wait_out(1, o_o)

            @pl.when(g + 2 < ngrp)
            def _():
                fire(g + 2, 0, s_e)
            drain_extract(g + 1, 1, s_o)
            write_out(g + 1, 1, o_o)

        wait_out(0, o_e)
        wait_out(1, o_o)

    return k(emb, idx3)


def _tc_mlp(x, W1, b1, W2, b2):
    n, d = x.shape
    blk = 4096

    def body(x_ref, w1_ref, b1_ref, w2_ref, b2_ref, o_ref):
        h = jnp.maximum(
            jnp.dot(x_ref[...], w1_ref[...], preferred_element_type=jnp.float32)
            + b1_ref[...], 0.0)
        o_ref[...] = (
            jnp.dot(h, w2_ref[...], preferred_element_type=jnp.float32)
            + b2_ref[...])

    return pl.pallas_call(
        body,
        grid=(n // blk,),
        in_specs=[
            pl.BlockSpec((blk, d), lambda i: (i, 0)),
            pl.BlockSpec((d, d), lambda i: (0, 0)),
            pl.BlockSpec((1, d), lambda i: (0, 0)),
            pl.BlockSpec((d, d), lambda i: (0, 0)),
            pl.BlockSpec((1, d), lambda i: (0, 0)),
        ],
        out_specs=pl.BlockSpec((blk, d), lambda i: (i, 0)),
        out_shape=jax.ShapeDtypeStruct((n, d), jnp.float32),
    )(x, W1, b1.reshape(1, d), W2, b2.reshape(1, d))


def kernel(class_indices, embedding, W1, b1, W2, b2):
    if class_indices.ndim == 1:
        class_indices = class_indices[:, None]
    q, b = class_indices.shape
    n = q * b
    per_w = n // NW
    idx3 = class_indices.reshape(NW, per_w // 128, 128).astype(jnp.int32)
    gathered = _sc_gather(embedding, idx3)
    out = _tc_mlp(gathered, W1, b1, W2, b2)
    return out.reshape(q, b, D)
